# baseline (speedup 1.0000x reference)
import functools

import jax
import jax.numpy as jnp
from jax import lax
from jax.experimental import pallas as pl
from jax.experimental.pallas import tpu as pltpu
from jax.experimental.pallas import tpu_sc as plsc

NC = 2
NS = 16
NW = NC * NS
L = 16


def _matmul(x, w):
    n, din = x.shape
    dout = w.shape[1]
    bm = 1000
    assert n % bm == 0

    def body(x_ref, w_ref, o_ref):
        o_ref[...] = jnp.dot(x_ref[...], w_ref[...],
                             preferred_element_type=jnp.float32)

    return pl.pallas_call(
        body,
        out_shape=jax.ShapeDtypeStruct((n, dout), jnp.float32),
        grid=(n // bm,),
        in_specs=[
            pl.BlockSpec((bm, din), lambda i: (i, 0)),
            pl.BlockSpec((din, dout), lambda i: (0, 0)),
        ],
        out_specs=pl.BlockSpec((bm, dout), lambda i: (i, 0)),
    )(x, w)


def _combine(partials, bias2d, n):
    d = partials.shape[2]
    bm = 1000
    assert n % bm == 0

    def body(p_ref, b_ref, o_ref):
        o_ref[...] = p_ref[0] + p_ref[1] + b_ref[...]

    return pl.pallas_call(
        body,
        out_shape=jax.ShapeDtypeStruct((n, d), jnp.float32),
        grid=(n // bm,),
        in_specs=[
            pl.BlockSpec((2, bm, d), lambda i: (0, i, 0)),
            pl.BlockSpec((1, d), lambda i: (0, 0)),
        ],
        out_specs=pl.BlockSpec((bm, d), lambda i: (i, 0)),
    )(partials, bias2d)


def _sc_scatter(support, dst, src, adj):
    n, d = support.shape
    e = dst.shape[0]
    assert d == 8 * L
    ep = e // NW
    assert ep * NW == e
    C = 80
    nchunk = ep // C
    assert nchunk * C == ep
    SG = 4
    nsg = (nchunk + SG - 1) // SG
    npc = nsg * SG
    np_ = ((n + C * NS - 1) // (C * NS)) * (C * NS)
    rows_per_tile = np_ // NS
    nz = rows_per_tile // C
    assert nz * C == rows_per_tile

    src3 = src.reshape(NW, nchunk, 1, C)
    dst3 = dst.reshape(NW, nchunk, 1, C)
    meta = jnp.concatenate([src3, dst3], axis=2)
    meta = jnp.pad(meta, ((0, 0), (0, npc - nchunk), (0, 0), (0, 0)))
    adj4 = adj.reshape(NW, nchunk, 1, C)
    adj4 = jnp.pad(adj4, ((0, 0), (0, npc - nchunk), (0, 0), (0, 0)))

    mesh = plsc.VectorSubcoreMesh(core_axis_name="c", subcore_axis_name="s",
                                  num_cores=NC, num_subcores=NS)

    @functools.partial(
        pl.kernel,
        out_type=jax.ShapeDtypeStruct((NC, np_, d), jnp.float32),
        mesh=mesh,
        scratch_types=[
            pltpu.VMEM_SHARED((np_, d), jnp.float32),
            pltpu.VMEM((3, SG, 2, C), jnp.int32),
            pltpu.VMEM((3, SG, 1, C), jnp.float32),
            pltpu.VMEM((2, C, d), jnp.float32),
            pltpu.VMEM((2, C, d), jnp.float32),
            pltpu.SemaphoreType.DMA,
            pltpu.SemaphoreType.DMA((2,)),
            pltpu.SemaphoreType.DMA((2,)),
        ],
    )
    def scatter_kernel(sup_hbm, meta_hbm, adj_hbm, out_hbm,
                       acc_sh, meta_v, adj_v, gbuf, sbuf, isem, gsem, ssem):
        cid = lax.axis_index("c")
        sid = lax.axis_index("s")
        wid = sid * NC + cid

        zero = jnp.zeros((L,), jnp.float32)

        def zrow(i, carry):
            for f in range(d // L):
                gbuf[0, i, pl.ds(f * L, L)] = zero
            return carry

        lax.fori_loop(0, C, zrow, 0)
        for j in range(nz):
            pltpu.sync_copy(gbuf.at[0],
                            acc_sh.at[pl.ds(sid * rows_per_tile + j * C, C)])
        plsc.subcore_barrier()

        for s0 in range(2):
            pltpu.sync_copy(meta_hbm.at[wid, pl.ds(s0 * SG, SG)],
                            meta_v.at[s0])
            pltpu.sync_copy(adj_hbm.at[wid, pl.ds(s0 * SG, SG)],
                            adj_v.at[s0])
        for g0 in range(2):
            pltpu.async_copy(sup_hbm.at[meta_v.at[0, g0, 0]], gbuf.at[g0],
                             gsem.at[g0])

        def chunk_iter(g, carry):
            bg = lax.rem(g, 2)
            sg_ = lax.div(g, SG)
            ms = lax.rem(sg_, 3)
            mb = lax.rem(g, SG)
            at_boundary = lax.rem(g, SG) == 2

            @pl.when(jnp.logical_and(at_boundary, g > 2))
            def _():
                pltpu.make_async_copy(meta_hbm.at[wid, pl.ds(0, SG)],
                                      meta_v.at[0], isem).wait()
                pltpu.make_async_copy(adj_hbm.at[wid, pl.ds(0, SG)],
                                      adj_v.at[0], isem).wait()

            @pl.when(jnp.logical_and(at_boundary, sg_ + 2 < nsg))
            def _():
                msl = lax.rem(sg_ + 2, 3)
                pltpu.async_copy(
                    meta_hbm.at[wid, pl.ds((sg_ + 2) * SG, SG)],
                    meta_v.at[msl], isem)
                pltpu.async_copy(
                    adj_hbm.at[wid, pl.ds((sg_ + 2) * SG, SG)],
                    adj_v.at[msl], isem)

            pltpu.make_async_copy(sup_hbm.at[meta_v.at[ms, mb, 0]],
                                  gbuf.at[bg], gsem.at[bg]).wait()

            @pl.when(g >= 2)
            def _():
                pltpu.make_async_copy(sbuf.at[bg],
                                      acc_sh.at[meta_v.at[ms, mb, 1]],
                                      ssem.at[bg]).wait()

            def sgroup(gi, ic):
                avec = adj_v[ms, mb, 0, pl.ds(gi * L, L)]
                for lane in range(L):
                    ab = avec[lane]
                    ei = gi * L + lane
                    for f in range(d // L):
                        sl = pl.ds(f * L, L)
                        sbuf[bg, ei, sl] = gbuf[bg, ei, sl] * ab
                return ic

            lax.fori_loop(0, C // L, sgroup, 0)

            pltpu.async_copy(sbuf.at[bg], acc_sh.at[meta_v.at[ms, mb, 1]],
                             ssem.at[bg], add=True)

            @pl.when(g + 2 < nchunk)
            def _():
                g2 = g + 2
                sg2 = lax.div(g2, SG)
                ms2 = lax.rem(sg2, 3)
                mb2 = lax.rem(g2, SG)
                pltpu.async_copy(sup_hbm.at[meta_v.at[ms2, mb2, 0]],
                                 gbuf.at[bg], gsem.at[bg])
            return carry

        lax.fori_loop(0, nchunk, chunk_iter, 0)

        for b in range(2):
            pltpu.make_async_copy(sbuf.at[b], acc_sh.at[meta_v.at[0, 0, 1]],
                                  ssem.at[b]).wait()
        plsc.subcore_barrier()

        pltpu.sync_copy(
            acc_sh.at[pl.ds(sid * rows_per_tile, rows_per_tile)],
            out_hbm.at[cid, pl.ds(sid * rows_per_tile, rows_per_tile)])

    return scatter_kernel(support, meta, adj4)


def kernel(input, edge_index, adj_values, weight, bias):
    support = _matmul(input, weight)
    partials = _sc_scatter(support, edge_index[0], edge_index[1], adj_values)
    return _combine(partials, bias.reshape(1, -1), input.shape[0])


# speedup vs baseline: 1.9919x; 1.9919x over previous
import functools

import jax
import jax.numpy as jnp
from jax import lax
from jax.experimental import pallas as pl
from jax.experimental.pallas import tpu as pltpu
from jax.experimental.pallas import tpu_sc as plsc

NC = 2
NS = 16
NW = NC * NS
L = 16


def _combine_matmul(partials, w, bias2d, n):
    din = partials.shape[2]
    dout = w.shape[1]
    bm = 1000
    assert n % bm == 0

    def body(p_ref, w_ref, b_ref, o_ref):
        acc = p_ref[0] + p_ref[1]
        o_ref[...] = jnp.dot(acc, w_ref[...],
                             preferred_element_type=jnp.float32) + b_ref[...]

    return pl.pallas_call(
        body,
        out_shape=jax.ShapeDtypeStruct((n, dout), jnp.float32),
        grid=(n // bm,),
        in_specs=[
            pl.BlockSpec((2, bm, din), lambda i: (0, i, 0)),
            pl.BlockSpec((din, dout), lambda i: (0, 0)),
            pl.BlockSpec((1, dout), lambda i: (0, 0)),
        ],
        out_specs=pl.BlockSpec((bm, dout), lambda i: (i, 0)),
    )(partials, w, bias2d)


def _sc_scatter(support, dst, src, adj):
    n, d = support.shape
    e = dst.shape[0]
    assert d == 8 * L
    ep = e // NW
    assert ep * NW == e
    C = 80
    NB = 4
    nchunk = ep // C
    assert nchunk * C == ep
    ngroup = (nchunk - 1) // NB
    assert ngroup * NB + 1 == nchunk
    np_ = ((n + C * NS - 1) // (C * NS)) * (C * NS)
    rows_per_tile = np_ // NS
    nz = rows_per_tile // C
    assert nz * C == rows_per_tile

    src3 = src.reshape(NW, nchunk, 1, C)
    dst3 = dst.reshape(NW, nchunk, 1, C)
    meta = jnp.concatenate([src3, dst3], axis=2)
    adj3 = adj.reshape(NW, nchunk, 1, C)

    mesh = plsc.VectorSubcoreMesh(core_axis_name="c", subcore_axis_name="s",
                                  num_cores=NC, num_subcores=NS)

    @functools.partial(
        pl.kernel,
        out_type=jax.ShapeDtypeStruct((NC, np_, d), jnp.float32),
        mesh=mesh,
        scratch_types=[
            pltpu.VMEM_SHARED((np_, d), jnp.float32),
            pltpu.VMEM((2, NB, 2, C), jnp.int32),
            pltpu.VMEM((2, NB, 1, C), jnp.float32),
            pltpu.VMEM((NB, C, d), jnp.float32),
            pltpu.SemaphoreType.DMA,
            pltpu.SemaphoreType.DMA((NB,)),
            pltpu.SemaphoreType.DMA((NB,)),
        ],
    )
    def scatter_kernel(sup_hbm, meta_hbm, adj_hbm, out_hbm,
                       acc_sh, meta_v, adj_v, rows_v, isem, gsem, ssem):
        cid = lax.axis_index("c")
        sid = lax.axis_index("s")
        wid = sid * NC + cid

        zero = jnp.zeros((L,), jnp.float32)

        def zrow(i, carry):
            for f in range(d // L):
                rows_v[0, i, pl.ds(f * L, L)] = zero
            return carry

        lax.fori_loop(0, C, zrow, 0)
        for j in range(nz):
            pltpu.sync_copy(rows_v.at[0],
                            acc_sh.at[pl.ds(sid * rows_per_tile + j * C, C)])
        plsc.subcore_barrier()

        pltpu.sync_copy(meta_hbm.at[wid, pl.ds(0, NB)], meta_v.at[0])
        pltpu.sync_copy(adj_hbm.at[wid, pl.ds(0, NB)], adj_v.at[0])
        pltpu.async_copy(meta_hbm.at[wid, pl.ds(NB, NB)], meta_v.at[1], isem)
        pltpu.async_copy(adj_hbm.at[wid, pl.ds(NB, NB)], adj_v.at[1], isem)
        for b in range(NB):
            pltpu.async_copy(sup_hbm.at[meta_v.at[0, b, 0]], rows_v.at[b],
                             gsem.at[b])

        def scale_chunk(rb, ms, mb):
            def sgroup(gi, ic):
                avec = adj_v[ms, mb, 0, pl.ds(gi * L, L)]
                for lane in range(L):
                    ab = avec[lane]
                    ei = gi * L + lane
                    for f in range(d // L):
                        sl = pl.ds(f * L, L)
                        rb[ei, sl] = rb[ei, sl] * ab
                return ic

            lax.fori_loop(0, C // L, sgroup, 0)

        def group_iter(G, carry):
            s = lax.rem(G, 2)
            for b in range(NB):
                rb = rows_v.at[b]
                pltpu.make_async_copy(sup_hbm.at[meta_v.at[s, b, 0]], rb,
                                      gsem.at[b]).wait()
                scale_chunk(rb, s, b)
                pltpu.async_copy(rb, acc_sh.at[meta_v.at[s, b, 1]],
                                 ssem.at[b], add=True)

            @pl.when(G < ngroup - 1)
            def _():
                sn = 1 - s
                for b in range(NB):
                    pltpu.make_async_copy(rows_v.at[b],
                                          acc_sh.at[meta_v.at[s, b, 1]],
                                          ssem.at[b]).wait()
                pltpu.make_async_copy(meta_hbm.at[wid, pl.ds(0, NB)],
                                      meta_v.at[sn], isem).wait()
                pltpu.make_async_copy(adj_hbm.at[wid, pl.ds(0, NB)],
                                      adj_v.at[sn], isem).wait()

                @pl.when(G < ngroup - 2)
                def _():
                    pltpu.async_copy(
                        meta_hbm.at[wid, pl.ds((G + 2) * NB, NB)],
                        meta_v.at[s], isem)
                    pltpu.async_copy(
                        adj_hbm.at[wid, pl.ds((G + 2) * NB, NB)],
                        adj_v.at[s], isem)

                for b in range(NB):
                    pltpu.async_copy(sup_hbm.at[meta_v.at[sn, b, 0]],
                                     rows_v.at[b], gsem.at[b])
            return carry

        lax.fori_loop(0, ngroup, group_iter, 0)

        sl_ = (ngroup - 1) % 2
        for b in range(NB):
            pltpu.make_async_copy(rows_v.at[b],
                                  acc_sh.at[meta_v.at[sl_, b, 1]],
                                  ssem.at[b]).wait()
        pltpu.sync_copy(meta_hbm.at[wid, pl.ds(nchunk - 1, 1)],
                        meta_v.at[0, pl.ds(0, 1)])
        pltpu.sync_copy(adj_hbm.at[wid, pl.ds(nchunk - 1, 1)],
                        adj_v.at[0, pl.ds(0, 1)])
        rb = rows_v.at[0]
        pltpu.async_copy(sup_hbm.at[meta_v.at[0, 0, 0]], rb,
                         gsem.at[0]).wait()
        scale_chunk(rb, 0, 0)
        pltpu.sync_copy(rb, acc_sh.at[meta_v.at[0, 0, 1]], add=True)

        plsc.subcore_barrier()
        pltpu.sync_copy(
            acc_sh.at[pl.ds(sid * rows_per_tile, rows_per_tile)],
            out_hbm.at[cid, pl.ds(sid * rows_per_tile, rows_per_tile)])

    return scatter_kernel(support, meta, adj3)


def kernel(input, edge_index, adj_values, weight, bias):
    partials = _sc_scatter(input, edge_index[0], edge_index[1], adj_values)
    return _combine_matmul(partials, weight, bias.reshape(1, -1),
                           input.shape[0])


# speedup vs baseline: 2.3853x; 1.1975x over previous
import functools

import jax
import jax.numpy as jnp
from jax import lax
from jax.experimental import pallas as pl
from jax.experimental.pallas import tpu as pltpu
from jax.experimental.pallas import tpu_sc as plsc

NC = 2
NS = 16
NW = NC * NS
L = 16


def _combine_matmul(partials, w, bias2d, n):
    din = partials.shape[2]
    dout = w.shape[1]
    bm = 1000
    assert n % bm == 0

    def body(p_ref, w_ref, b_ref, o_ref):
        acc = p_ref[0] + p_ref[1]
        o_ref[...] = jnp.dot(acc, w_ref[...],
                             preferred_element_type=jnp.float32) + b_ref[...]

    return pl.pallas_call(
        body,
        out_shape=jax.ShapeDtypeStruct((n, dout), jnp.float32),
        grid=(n // bm,),
        in_specs=[
            pl.BlockSpec((2, bm, din), lambda i: (0, i, 0)),
            pl.BlockSpec((din, dout), lambda i: (0, 0)),
            pl.BlockSpec((1, dout), lambda i: (0, 0)),
        ],
        out_specs=pl.BlockSpec((bm, dout), lambda i: (i, 0)),
    )(partials, w, bias2d)


def _sc_scatter(support, dst, src, adj):
    n, d = support.shape
    e = dst.shape[0]
    assert d == 8 * L
    ep = e // NW
    assert ep * NW == e
    C = 80
    nchunk = ep // C
    assert nchunk * C == ep
    npair = nchunk // 2
    npp = npair + 2
    MD = 4
    np_ = ((n + C * NS - 1) // (C * NS)) * (C * NS)
    rows_per_tile = np_ // NS
    nz = rows_per_tile // C
    assert nz * C == rows_per_tile

    src3 = src.reshape(NW, nchunk, 1, C)
    dst3 = dst.reshape(NW, nchunk, 1, C)
    meta = jnp.concatenate([src3, dst3], axis=2)
    meta = jnp.pad(meta, ((0, 0), (0, 2 * npp - nchunk), (0, 0), (0, 0)))
    meta = meta.reshape(NW, npp, 2, 2, C)
    adj4 = adj.reshape(NW, nchunk, 1, C)
    adj4 = jnp.pad(adj4, ((0, 0), (0, 2 * npp - nchunk), (0, 0), (0, 0)))
    adj4 = adj4.reshape(NW, npp, 2, 1, C)

    mesh = plsc.VectorSubcoreMesh(core_axis_name="c", subcore_axis_name="s",
                                  num_cores=NC, num_subcores=NS)

    @functools.partial(
        pl.kernel,
        out_type=jax.ShapeDtypeStruct((NC, np_, d), jnp.float32),
        mesh=mesh,
        scratch_types=[
            pltpu.VMEM_SHARED((np_, d), jnp.float32),
            pltpu.VMEM((MD, 2, 2, C), jnp.int32),
            pltpu.VMEM((MD, 2, 1, C), jnp.float32),
            pltpu.VMEM((2, C, d), jnp.float32),
            pltpu.VMEM((2, C, d), jnp.float32),
            pltpu.SemaphoreType.DMA,
            pltpu.SemaphoreType.DMA((2,)),
            pltpu.SemaphoreType.DMA((2,)),
        ],
    )
    def scatter_kernel(sup_hbm, meta_hbm, adj_hbm, out_hbm,
                       acc_sh, meta_v, adj_v, gbuf, sbuf, isem, gsem, ssem):
        cid = lax.axis_index("c")
        sid = lax.axis_index("s")
        wid = sid * NC + cid

        zero = jnp.zeros((L,), jnp.float32)

        def zrow(i, carry):
            for f in range(d // L):
                gbuf[0, i, pl.ds(f * L, L)] = zero
            return carry

        lax.fori_loop(0, C, zrow, 0)
        for j in range(nz):
            pltpu.sync_copy(gbuf.at[0],
                            acc_sh.at[pl.ds(sid * rows_per_tile + j * C, C)])
        plsc.subcore_barrier()

        def scale_chunk(par, md):
            def sgroup(gi, ic):
                avec = adj_v[md, par, 0, pl.ds(gi * L, L)]
                for lane in range(L):
                    ab = avec[lane]
                    ei = gi * L + lane
                    for f in range(d // L):
                        sl = pl.ds(f * L, L)
                        sbuf[par, ei, sl] = gbuf[par, ei, sl] * ab
                return ic

            lax.fori_loop(0, C // L, sgroup, 0)

        for p0 in range(2):
            pltpu.sync_copy(meta_hbm.at[wid, p0], meta_v.at[p0])
            pltpu.sync_copy(adj_hbm.at[wid, p0], adj_v.at[p0])
        pltpu.async_copy(meta_hbm.at[wid, 2], meta_v.at[2], isem)
        pltpu.async_copy(adj_hbm.at[wid, 2], adj_v.at[2], isem)
        for par in range(2):
            pltpu.async_copy(sup_hbm.at[meta_v.at[0, par, 0]], gbuf.at[par],
                             gsem.at[par])

        def pair_iter(t, carry):
            md = lax.rem(t, MD)
            mdn = lax.rem(t + 1, MD)

            @pl.when(jnp.logical_and(t >= 1, t + 1 <= npair - 1))
            def _():
                pltpu.make_async_copy(meta_hbm.at[wid, 0],
                                      meta_v.at[0], isem).wait()
                pltpu.make_async_copy(adj_hbm.at[wid, 0],
                                      adj_v.at[0], isem).wait()

            for par in range(2):
                g = 2 * t + par
                pltpu.make_async_copy(sup_hbm.at[meta_v.at[md, par, 0]],
                                      gbuf.at[par], gsem.at[par]).wait()

                @pl.when(t >= 1)
                def _():
                    pltpu.make_async_copy(sbuf.at[par],
                                          acc_sh.at[meta_v.at[md, par, 1]],
                                          ssem.at[par]).wait()

                scale_chunk(par, md)

                pltpu.async_copy(sbuf.at[par],
                                 acc_sh.at[meta_v.at[md, par, 1]],
                                 ssem.at[par], add=True)

                @pl.when(t + 1 <= npair - 1)
                def _():
                    pltpu.async_copy(sup_hbm.at[meta_v.at[mdn, par, 0]],
                                     gbuf.at[par], gsem.at[par])

            @pl.when(t + 3 <= npair - 1)
            def _():
                msl = lax.rem(t + 3, MD)
                pltpu.async_copy(meta_hbm.at[wid, t + 3], meta_v.at[msl],
                                 isem)
                pltpu.async_copy(adj_hbm.at[wid, t + 3], adj_v.at[msl],
                                 isem)
            return carry

        lax.fori_loop(0, npair, pair_iter, 0)

        for b in range(2):
            pltpu.make_async_copy(sbuf.at[b], acc_sh.at[meta_v.at[0, b, 1]],
                                  ssem.at[b]).wait()

        pltpu.sync_copy(meta_hbm.at[wid, npair], meta_v.at[0])
        pltpu.sync_copy(adj_hbm.at[wid, npair], adj_v.at[0])
        pltpu.async_copy(sup_hbm.at[meta_v.at[0, 0, 0]], gbuf.at[0],
                         gsem.at[0]).wait()
        scale_chunk(0, 0)
        pltpu.sync_copy(sbuf.at[0], acc_sh.at[meta_v.at[0, 0, 1]], add=True)

        plsc.subcore_barrier()
        pltpu.sync_copy(
            acc_sh.at[pl.ds(sid * rows_per_tile, rows_per_tile)],
            out_hbm.at[cid, pl.ds(sid * rows_per_tile, rows_per_tile)])

    return scatter_kernel(support, meta, adj4)


def kernel(input, edge_index, adj_values, weight, bias):
    partials = _sc_scatter(input, edge_index[0], edge_index[1], adj_values)
    return _combine_matmul(partials, weight, bias.reshape(1, -1),
                           input.shape[0])


# speedup vs baseline: 2.4106x; 1.0106x over previous
import functools

import jax
import jax.numpy as jnp
from jax import lax
from jax.experimental import pallas as pl
from jax.experimental.pallas import tpu as pltpu
from jax.experimental.pallas import tpu_sc as plsc

NC = 2
NS = 16
NW = NC * NS
L = 16


def _combine_matmul(partials, w, bias2d, n):
    din = partials.shape[2]
    dout = w.shape[1]
    bm = 1000
    assert n % bm == 0

    def body(p_ref, w_ref, b_ref, o_ref):
        acc = p_ref[0] + p_ref[1]
        o_ref[...] = jnp.dot(acc, w_ref[...],
                             preferred_element_type=jnp.float32) + b_ref[...]

    return pl.pallas_call(
        body,
        out_shape=jax.ShapeDtypeStruct((n, dout), jnp.float32),
        grid=(n // bm,),
        in_specs=[
            pl.BlockSpec((2, bm, din), lambda i: (0, i, 0)),
            pl.BlockSpec((din, dout), lambda i: (0, 0)),
            pl.BlockSpec((1, dout), lambda i: (0, 0)),
        ],
        out_specs=pl.BlockSpec((bm, dout), lambda i: (i, 0)),
    )(partials, w, bias2d)


def _sc_scatter(support, dst, src, adj):
    n, d = support.shape
    e = dst.shape[0]
    assert d == 8 * L
    ep = e // NW
    assert ep * NW == e
    C = 80
    nchunk = ep // C
    assert nchunk * C == ep
    npair = nchunk // 2
    npp = npair + 2
    MD = 4
    np_ = ((n + C * NS - 1) // (C * NS)) * (C * NS)
    rows_per_tile = np_ // NS
    nz = rows_per_tile // C
    assert nz * C == rows_per_tile

    src3 = src.reshape(NW, nchunk, 1, C)
    dst3 = dst.reshape(NW, nchunk, 1, C)
    meta = jnp.concatenate([src3, dst3], axis=2)
    meta = jnp.pad(meta, ((0, 0), (0, 2 * npp - nchunk), (0, 0), (0, 0)))
    meta = meta.reshape(NW, npp, 2, 2, C)
    adj4 = adj.reshape(NW, nchunk, 1, C)
    adj4 = jnp.pad(adj4, ((0, 0), (0, 2 * npp - nchunk), (0, 0), (0, 0)))
    adj4 = adj4.reshape(NW, npp, 2, 1, C)

    mesh = plsc.VectorSubcoreMesh(core_axis_name="c", subcore_axis_name="s",
                                  num_cores=NC, num_subcores=NS)

    @functools.partial(
        pl.kernel,
        out_type=jax.ShapeDtypeStruct((NC, np_, d), jnp.float32),
        mesh=mesh,
        scratch_types=[
            pltpu.VMEM_SHARED((np_, d), jnp.float32),
            pltpu.VMEM((MD, 2, 2, C), jnp.int32),
            pltpu.VMEM((MD, 2, 1, C), jnp.float32),
            pltpu.VMEM((2, C, d), jnp.float32),
            pltpu.VMEM((2, C, d), jnp.float32),
            pltpu.SemaphoreType.DMA,
            pltpu.SemaphoreType.DMA((2,)),
            pltpu.SemaphoreType.DMA((2,)),
        ],
    )
    def scatter_kernel(sup_hbm, meta_hbm, adj_hbm, out_hbm,
                       acc_sh, meta_v, adj_v, gbuf, sbuf, isem, gsem, ssem):
        cid = lax.axis_index("c")
        sid = lax.axis_index("s")
        wid = sid * NC + cid


        def scale_chunk(par, md):
            def sgroup(gi, ic):
                avec = adj_v[md, par, 0, pl.ds(gi * L, L)]
                for lane in range(L):
                    ab = avec[lane]
                    ei = gi * L + lane
                    for f in range(d // L):
                        sl = pl.ds(f * L, L)
                        sbuf[par, ei, sl] = gbuf[par, ei, sl] * ab
                return ic

            lax.fori_loop(0, C // L, sgroup, 0)

        for p0 in range(2):
            pltpu.sync_copy(meta_hbm.at[wid, p0], meta_v.at[p0])
            pltpu.sync_copy(adj_hbm.at[wid, p0], adj_v.at[p0])
        pltpu.async_copy(meta_hbm.at[wid, 2], meta_v.at[2], isem)
        pltpu.async_copy(adj_hbm.at[wid, 2], adj_v.at[2], isem)
        for par in range(2):
            pltpu.async_copy(sup_hbm.at[meta_v.at[0, par, 0]], gbuf.at[par],
                             gsem.at[par])

        zero = jnp.zeros((L,), jnp.float32)

        def zrow(i, carry):
            for f in range(d // L):
                sbuf[0, i, pl.ds(f * L, L)] = zero
            return carry

        lax.fori_loop(0, C, zrow, 0)
        for j in range(nz):
            pltpu.sync_copy(sbuf.at[0],
                            acc_sh.at[pl.ds(sid * rows_per_tile + j * C, C)])
        plsc.subcore_barrier()

        def pair_iter(t, carry):
            md = lax.rem(t, MD)
            mdn = lax.rem(t + 1, MD)

            @pl.when(jnp.logical_and(t >= 1, t + 1 <= npair - 1))
            def _():
                pltpu.make_async_copy(meta_hbm.at[wid, 0],
                                      meta_v.at[0], isem).wait()
                pltpu.make_async_copy(adj_hbm.at[wid, 0],
                                      adj_v.at[0], isem).wait()

            for par in range(2):
                g = 2 * t + par
                pltpu.make_async_copy(sup_hbm.at[meta_v.at[md, par, 0]],
                                      gbuf.at[par], gsem.at[par]).wait()

                @pl.when(t >= 1)
                def _():
                    pltpu.make_async_copy(sbuf.at[par],
                                          acc_sh.at[meta_v.at[md, par, 1]],
                                          ssem.at[par]).wait()

                scale_chunk(par, md)

                pltpu.async_copy(sbuf.at[par],
                                 acc_sh.at[meta_v.at[md, par, 1]],
                                 ssem.at[par], add=True)

                @pl.when(t + 1 <= npair - 1)
                def _():
                    pltpu.async_copy(sup_hbm.at[meta_v.at[mdn, par, 0]],
                                     gbuf.at[par], gsem.at[par])

            @pl.when(t + 3 <= npair - 1)
            def _():
                msl = lax.rem(t + 3, MD)
                pltpu.async_copy(meta_hbm.at[wid, t + 3], meta_v.at[msl],
                                 isem)
                pltpu.async_copy(adj_hbm.at[wid, t + 3], adj_v.at[msl],
                                 isem)
            return carry

        lax.fori_loop(0, npair, pair_iter, 0)

        for b in range(2):
            pltpu.make_async_copy(sbuf.at[b], acc_sh.at[meta_v.at[0, b, 1]],
                                  ssem.at[b]).wait()

        pltpu.sync_copy(meta_hbm.at[wid, npair], meta_v.at[0])
        pltpu.sync_copy(adj_hbm.at[wid, npair], adj_v.at[0])
        pltpu.async_copy(sup_hbm.at[meta_v.at[0, 0, 0]], gbuf.at[0],
                         gsem.at[0]).wait()
        scale_chunk(0, 0)
        pltpu.sync_copy(sbuf.at[0], acc_sh.at[meta_v.at[0, 0, 1]], add=True)

        plsc.subcore_barrier()
        pltpu.sync_copy(
            acc_sh.at[pl.ds(sid * rows_per_tile, rows_per_tile)],
            out_hbm.at[cid, pl.ds(sid * rows_per_tile, rows_per_tile)])

    return scatter_kernel(support, meta, adj4)


def kernel(input, edge_index, adj_values, weight, bias):
    partials = _sc_scatter(input, edge_index[0], edge_index[1], adj_values)
    return _combine_matmul(partials, weight, bias.reshape(1, -1),
                           input.shape[0])
